# Initial kernel scaffold; baseline (speedup 1.0000x reference)
import jax
import jax.numpy as jnp
from jax.experimental import pallas as pl


def kernel(inputs, weights, biases, s0_rows, s0_cols, s0_vals, s1_rows, s1_cols, s1_vals):
    raise NotImplementedError("write your pallas kernel here")


# speedup vs baseline: 6.4497x; 6.4497x over previous
import functools

import jax
import jax.numpy as jnp
from jax import lax
from jax.experimental import pallas as pl
from jax.experimental.pallas import tpu as pltpu
from jax.experimental.pallas import tpu_sc as plsc

N = 4096
FEAT = 512
NSUB = 32
ROWS_PER_SUB = N // NSUB
CHUNK = 256
G = 32
NG = CHUNK // G
LANES = 16
NJ = FEAT // LANES
PBLK = 32


def _spmm_body(scale, has_prev,
               z_hbm, prev_hbm, rows_hbm, cols_hbm, vals_hbm, start_hbm,
               out_hbm,
               start_v, meta_r, meta_c, meta_v, gbuf, acc, pbuf,
               sem0, sem1):
    c = lax.axis_index("c")
    s = lax.axis_index("s")
    wid = c * 16 + s
    base = wid * ROWS_PER_SUB

    pltpu.sync_copy(start_hbm, start_v)
    sv = start_v[pl.ds(wid, LANES)]
    e0 = sv[0]
    e1 = sv[1]
    e0a = jnp.bitwise_and(e0, -16)
    nchunks = (e1 - e0a + CHUNK - 1) // CHUNK

    zv = jnp.zeros((LANES,), jnp.float32)

    @plsc.parallel_loop(0, ROWS_PER_SUB)
    def _(r):
        for j in range(NJ):
            acc[r, pl.ds(j * LANES, LANES)] = zv

    sems = [sem0, sem1]

    def gather_copy(gi, buf):
        return pltpu.make_async_copy(
            z_hbm.at[meta_c.at[pl.ds(gi * G, G)]],
            gbuf.at[buf],
            sems[buf],
        )

    def chunk_body(ci, carry):
        cb = pl.multiple_of(e0a + ci * CHUNK, 16)
        pltpu.sync_copy(rows_hbm.at[pl.ds(cb, CHUNK)], meta_r.at[pl.ds(0, CHUNK)])
        pltpu.sync_copy(cols_hbm.at[pl.ds(cb, CHUNK)], meta_c)
        pltpu.sync_copy(vals_hbm.at[pl.ds(cb, CHUNK)], meta_v.at[pl.ds(0, CHUNK)])

        def make_carry(idx):
            rv = meta_r[pl.ds(idx, LANES)]
            vv16 = meta_v[pl.ds(idx, LANES)]
            rl = jnp.clip(rv[0] - base, 0, ROWS_PER_SUB - 1)
            eg = cb + idx
            ok = jnp.logical_and(eg >= e0, eg < e1)
            v = jnp.where(ok, vv16[0] * scale, 0.0)
            return rl, jnp.broadcast_to(v, (LANES,))

        gather_copy(0, 0).start()
        ecarry = make_carry(0)
        for gi in range(NG):
            if gi + 1 < NG:
                gather_copy(gi + 1, (gi + 1) % 2).start()
            gather_copy(gi, gi % 2).wait()
            buf = gi % 2

            def edge_body(i, ecarry, gi=gi, buf=buf):
                rl, vvb = ecarry
                idx = gi * G + i
                nxt = make_carry(idx + 1)
                for j0 in range(0, NJ, 8):
                    prods = [vvb * gbuf[buf, i, pl.ds((j0 + j) * LANES, LANES)]
                             for j in range(8)]
                    for j in range(8):
                        plsc.addupdate(
                            acc.at[rl, pl.ds((j0 + j) * LANES, LANES)], prods[j])
                return nxt

            ecarry = plsc.parallel_loop(0, G, unroll=2, carry=ecarry)(edge_body)
        return carry

    lax.fori_loop(0, nchunks, chunk_body, 0)

    if has_prev:
        for rb in range(ROWS_PER_SUB // PBLK):
            pltpu.sync_copy(prev_hbm.at[pl.ds(base + rb * PBLK, PBLK)], pbuf)

            @plsc.parallel_loop(0, PBLK)
            def _(r, rb=rb):
                for j in range(NJ):
                    sl = pl.ds(j * LANES, LANES)
                    acc[rb * PBLK + r, sl] = acc[rb * PBLK + r, sl] - pbuf[r, sl]

    pltpu.sync_copy(acc, out_hbm.at[pl.ds(base, ROWS_PER_SUB)])


@functools.cache
def _make_spmm(scale, has_prev):
    mesh = plsc.VectorSubcoreMesh(core_axis_name="c", subcore_axis_name="s")
    return pl.kernel(
        functools.partial(_spmm_body, scale, has_prev),
        out_type=jax.ShapeDtypeStruct((N, FEAT), jnp.float32),
        mesh=mesh,
        scratch_types=[
            pltpu.VMEM((64,), jnp.int32),
            pltpu.VMEM((CHUNK + LANES,), jnp.int32),
            pltpu.VMEM((CHUNK,), jnp.int32),
            pltpu.VMEM((CHUNK + LANES,), jnp.float32),
            pltpu.VMEM((2, G, FEAT), jnp.float32),
            pltpu.VMEM((ROWS_PER_SUB, FEAT), jnp.float32),
            pltpu.VMEM((PBLK, FEAT), jnp.float32),
            pltpu.SemaphoreType.DMA,
            pltpu.SemaphoreType.DMA,
        ],
    )


def _proj_kernel(z0, z1, z2, z3, z4, w, b, out):
    xs = [z[:, 0, 0, :] for z in (z0, z1, z2, z3, z4)]
    x = jnp.concatenate(xs, axis=1)
    acc = jnp.dot(x, w[:], preferred_element_type=jnp.float32)
    out[0] = acc + b[0]


def _project(z0, z1, z2, z3, z4, weights, biases):
    wp = weights.reshape(FEAT // 32, 5, 16).transpose(1, 0, 2).reshape(80, 16)
    zspec = pl.BlockSpec((N, 1, 1, 16), lambda bb: (0, bb, 0, 0))
    zs = [z.reshape(N, 32, 1, 16) for z in (z0, z1, z2, z3, z4)]
    return pl.pallas_call(
        _proj_kernel,
        grid=(32,),
        in_specs=[zspec] * 5 + [
            pl.BlockSpec((80, 16), lambda bb: (0, 0)),
            pl.BlockSpec((1, 16), lambda bb: (0, 0)),
        ],
        out_specs=pl.BlockSpec((1, N, 16), lambda bb: (bb, 0, 0)),
        out_shape=jax.ShapeDtypeStruct((32, N, 16), jnp.float32),
    )(*zs, wp, biases)


def _prep_support(r, c, v):
    nnz = r.shape[0]
    npad = (nnz // CHUNK + 2) * CHUNK
    pad = npad - nnz
    rp = jnp.concatenate([r.astype(jnp.int32), jnp.full((pad,), N, jnp.int32)])
    cp = jnp.concatenate([c.astype(jnp.int32), jnp.zeros((pad,), jnp.int32)])
    vp = jnp.concatenate([v, jnp.zeros((pad,), jnp.float32)])
    bnd = jnp.arange(NSUB + 1, dtype=jnp.int32) * ROWS_PER_SUB
    st = jnp.searchsorted(rp, bnd).astype(jnp.int32)
    st = jnp.concatenate([st, jnp.zeros((64 - NSUB - 1,), jnp.int32)])
    return rp, cp, vp, st


def kernel(inputs, weights, biases, s0_rows, s0_cols, s0_vals,
           s1_rows, s1_cols, s1_vals):
    batch, n, idim = inputs.shape
    z0 = jnp.transpose(inputs, (1, 0, 2)).reshape(n, batch * idim)
    r0, c0, v0, st0 = _prep_support(s0_rows, s0_cols, s0_vals)
    r1, c1, v1, st1 = _prep_support(s1_rows, s1_cols, s1_vals)
    spmm = _make_spmm(1.0, False)
    spmm_cheb = _make_spmm(2.0, True)
    z1 = spmm(z0, z0, r0, c0, v0, st0)
    z2 = spmm_cheb(z1, z0, r0, c0, v0, st0)
    z3 = spmm(z1, z1, r1, c1, v1, st1)
    z4 = spmm_cheb(z3, z1, r1, c1, v1, st1)
    return _project(z0, z1, z2, z3, z4, weights, biases)
